# baseline (speedup 1.0000x reference)
import functools

import jax
import jax.numpy as jnp
from jax.experimental import pallas as pl
from jax.experimental.pallas import tpu as pltpu
from jax.experimental.pallas import tpu_sc as plsc

NUM_EMBEDDINGS = 1024
EMBEDDING_DIM = 64
COMMITMENT_COST = 0.25


_BM = 2048


def _dist_argmin_body(xt_ref, w_ref, wm2_ref, idx_ref, loss_ref):
    w = w_ref[...]
    w2 = jnp.sum(w * w, axis=1, keepdims=True)
    loss_part = jnp.float32(0.0)
    mnegs = [
        jax.lax.dot_general(wm2_ref[...], xt_ref[p],
                            (((1,), (0,)), ((), ())),
                            preferred_element_type=jnp.float32)
        for p in range(_PLANES)
    ]
    for p in range(_PLANES):
        xt = xt_ref[p]
        x2 = jnp.sum(xt * xt, axis=0, keepdims=True)
        dist = (x2 + mnegs[p]) + w2
        minval = jnp.min(dist, axis=0, keepdims=True)
        iota = jax.lax.broadcasted_iota(
            jnp.int32, dist.shape, 0).astype(jnp.float32)
        cand = jnp.where(dist == minval, iota, jnp.float32(NUM_EMBEDDINGS))
        idxf = jnp.min(cand, axis=0, keepdims=True)
        idx_ref[0, p] = idxf.astype(jnp.int32).reshape(8, 128)
        loss_part += jnp.sum(minval)

    @pl.when(pl.program_id(0) == 0)
    def _init():
        loss_ref[...] = jnp.zeros_like(loss_ref)

    loss_ref[...] += loss_part.reshape(1, 1)


_PLANES = 2


def _dist_argmin(tokens_t, W, Wm2):
    B, D, K = tokens_t.shape
    n = B * K
    grid = (B // _PLANES,)
    idx, loss = pl.pallas_call(
        _dist_argmin_body,
        grid=grid,
        in_specs=[
            pl.BlockSpec((_PLANES, D, K), lambda i: (i, 0, 0)),
            pl.BlockSpec((NUM_EMBEDDINGS, EMBEDDING_DIM), lambda i: (0, 0)),
            pl.BlockSpec((NUM_EMBEDDINGS, EMBEDDING_DIM), lambda i: (0, 0)),
        ],
        out_specs=[
            pl.BlockSpec((1, _PLANES, 8, 128), lambda i: (i, 0, 0, 0)),
            pl.BlockSpec((1, 1), lambda i: (0, 0)),
        ],
        out_shape=[
            jax.ShapeDtypeStruct((B // _PLANES, _PLANES, 8, 128), jnp.int32),
            jax.ShapeDtypeStruct((1, 1), jnp.float32),
        ],
    )(tokens_t, W, Wm2)
    return idx, loss[0, 0]


_NW = 32
_CHUNK = 128
_ROW = 128


def _gather_rows(W_pad, idx4):
    steps, planes, _, _ = idx4.shape
    n = steps * planes * 1024
    b_per_w = n // _NW
    nch = b_per_w // _CHUNK
    mesh = plsc.VectorSubcoreMesh(core_axis_name="c", subcore_axis_name="s")

    @functools.partial(
        pl.kernel,
        mesh=mesh,
        out_type=jax.ShapeDtypeStruct((n, _ROW), jnp.float32),
        scratch_types=[
            pltpu.VMEM((nch, _CHUNK), jnp.int32),
            pltpu.VMEM((b_per_w, _ROW), jnp.float32),
            pltpu.VMEM_SHARED((NUM_EMBEDDINGS, _ROW), jnp.float32),
            pltpu.SemaphoreType.DMA,
        ],
    )
    def _k(table_hbm, idx_hbm, out_hbm, idx_v, rows_v, table_sp, sem):
        sid = jax.lax.axis_index("s")
        wid = sid * 2 + jax.lax.axis_index("c")
        base = wid * b_per_w
        @pl.when(sid == 0)
        def _load():
            pltpu.sync_copy(table_hbm, table_sp)

        g = wid // 2
        s = g // planes
        p = g % planes
        h = wid % 2
        pltpu.sync_copy(idx_hbm.at[s, p, pl.ds(h * nch, nch)], idx_v)
        plsc.subcore_barrier()
        gathers = [
            pltpu.async_copy(
                table_sp.at[idx_v.at[j]],
                rows_v.at[pl.ds(j * _CHUNK, _CHUNK)],
                sem,
            )
            for j in range(nch)
        ]
        for g in gathers:
            g.wait()
        pltpu.sync_copy(rows_v, out_hbm.at[pl.ds(base, b_per_w)])

    return _k(W_pad, idx4)


def kernel(tokens, W):
    B, K, D = tokens.shape
    n = B * K
    W_pad = jnp.pad(W, ((0, 0), (0, _ROW - D)))
    idx4, loss_sum = _dist_argmin(jnp.swapaxes(tokens, 1, 2), W, -2.0 * W)
    idx = idx4.reshape(n)
    q_st = _gather_rows(W_pad, idx4)[:, :D]
    vq_loss = (1.0 + COMMITMENT_COST) * loss_sum / (n * D)
    return (q_st.reshape(B, K, D), vq_loss, idx.reshape(B, K))
